# Initial kernel scaffold; baseline (speedup 1.0000x reference)
import functools

import jax
import jax.numpy as jnp
from jax import lax
from jax.experimental import pallas as pl
from jax.experimental.pallas import tpu as pltpu
from jax.experimental.pallas import tpu_sc as plsc

NC, NS, L = 2, 16, 16
NW = NC * NS
BLK = 128
ROW_UNROLL = 4


@functools.partial(jax.jit, static_argnames=("nq", "c", "h", "w"))
def _flosp_sc(t1, t2, t4, t8, px, py, fov, *, nq, c, h, w):
    qpw = nq // NW
    nblk = qpw // BLK
    shifts = (0, 1, 2, 3)
    ws = tuple(w >> s for s in shifts)
    hs = tuple(h >> s for s in shifts)
    padrow = tuple(hs[i] * ws[i] for i in range(4))
    cvec = c // L

    mesh = plsc.VectorSubcoreMesh(core_axis_name="c", subcore_axis_name="s")

    def body(t1h, t2h, t4h, t8h, pxh, pyh, fovh, outh,
             px_v, py_v, fov_v, i0, i1, i2, i3, r0, r1, r2, r3, sem):
        wid = lax.axis_index("s") * NC + lax.axis_index("c")
        qbase = wid * qpw
        pltpu.sync_copy(pxh.at[pl.ds(qbase, qpw)], px_v)
        pltpu.sync_copy(pyh.at[pl.ds(qbase, qpw)], py_v)
        pltpu.sync_copy(fovh.at[pl.ds(qbase, qpw)], fov_v)

        tables = (t1h, t2h, t4h, t8h)
        idxs = (i0, i1, i2, i3)
        rows = (r0, r1, r2, r3)

        def block(b, carry):
            for j in range(BLK // L):
                sl = pl.ds(b * BLK + j * L, L)
                x = px_v[sl]
                y = py_v[sl]
                m = fov_v[sl] > 0
                for si in range(4):
                    ix = lax.shift_right_logical(x, shifts[si])
                    iy = lax.shift_right_logical(y, shifts[si])
                    iid = iy * ws[si] + ix
                    iid = jnp.where(m, iid, padrow[si])
                    idxs[si][pl.ds(j * L, L)] = iid
            cps = [pltpu.async_copy(tables[si].at[idxs[si]], rows[si], sem)
                   for si in range(4)]
            for cp in cps:
                cp.wait()

            def acc(r, _):
                for u in range(ROW_UNROLL):
                    rr = r * ROW_UNROLL + u
                    for k in range(cvec):
                        slk = pl.ds(k * L, L)
                        v = ((r0[rr, slk] + r1[rr, slk])
                             + (r2[rr, slk] + r3[rr, slk]))
                        r0[rr, slk] = v
                return _

            lax.fori_loop(0, BLK // ROW_UNROLL, acc, 0)
            pltpu.sync_copy(r0, outh.at[pl.ds(qbase + b * BLK, BLK)])
            return carry

        lax.fori_loop(0, nblk, block, 0)

    run = pl.kernel(
        body,
        out_type=jax.ShapeDtypeStruct((nq, c), jnp.float32),
        mesh=mesh,
        scratch_types=[
            pltpu.VMEM((qpw,), jnp.int32),
            pltpu.VMEM((qpw,), jnp.int32),
            pltpu.VMEM((qpw,), jnp.int32),
            pltpu.VMEM((BLK,), jnp.int32),
            pltpu.VMEM((BLK,), jnp.int32),
            pltpu.VMEM((BLK,), jnp.int32),
            pltpu.VMEM((BLK,), jnp.int32),
            pltpu.VMEM((BLK, c), jnp.float32),
            pltpu.VMEM((BLK, c), jnp.float32),
            pltpu.VMEM((BLK, c), jnp.float32),
            pltpu.VMEM((BLK, c), jnp.float32),
            pltpu.SemaphoreType.DMA,
        ],
    )
    return run(t1, t2, t4, t8, px, py, fov)


def kernel(feat_s1, feat_s2, feat_s4, feat_s8, projected_pix, fov_mask):
    feats = (feat_s1, feat_s2, feat_s4, feat_s8)
    bs, num_cam, c, h, w = feat_s1.shape
    nq = projected_pix.shape[1]

    tables = []
    for f in feats:
        hw = f.shape[3] * f.shape[4]
        t = f.reshape(c, hw).T
        tables.append(jnp.concatenate([t, jnp.zeros((1, c), t.dtype)], axis=0))

    px = projected_pix[0, :, 0]
    py = projected_pix[0, :, 1]
    fov = fov_mask[0].astype(jnp.int32)

    y = _flosp_sc(*tables, px, py, fov, nq=nq, c=c, h=h, w=w)
    return y.T.reshape(bs, c, nq)


# speedup vs baseline: 305.1718x; 305.1718x over previous
import functools

import jax
import jax.numpy as jnp
from jax import lax
from jax.experimental import pallas as pl
from jax.experimental.pallas import tpu as pltpu
from jax.experimental.pallas import tpu_sc as plsc

NC, NS, L = 2, 16, 16
NW = NC * NS
BLK = 128
ROW_UNROLL = 4


@functools.partial(jax.jit, static_argnames=("nq", "c", "h", "w"))
def _flosp_sc(t1, t2, t4, t8, px, py, fov, *, nq, c, h, w):
    qpw = nq // NW
    nblk = qpw // BLK
    shifts = (0, 1, 2, 3)
    ws = tuple(w >> s for s in shifts)
    hs = tuple(h >> s for s in shifts)
    padrow = tuple(hs[i] * ws[i] for i in range(4))
    cvec = c // L

    mesh = plsc.VectorSubcoreMesh(core_axis_name="c", subcore_axis_name="s")

    def body(t1h, t2h, t4h, t8h, pxh, pyh, fovh, outh,
             px_v, py_v, fov_v, i0, i1, i2, i3, r0, r1, r2, r3, sem):
        wid = lax.axis_index("s") * NC + lax.axis_index("c")
        qbase = wid * qpw
        pltpu.sync_copy(pxh.at[pl.ds(qbase, qpw)], px_v)
        pltpu.sync_copy(pyh.at[pl.ds(qbase, qpw)], py_v)
        pltpu.sync_copy(fovh.at[pl.ds(qbase, qpw)], fov_v)

        tables = (t1h, t2h, t4h, t8h)
        idxs = (i0, i1, i2, i3)
        rows = (r0, r1, r2, r3)

        def block(b, carry):
            for j in range(BLK // L):
                sl = pl.ds(b * BLK + j * L, L)
                x = px_v[sl]
                y = py_v[sl]
                m = fov_v[sl] > 0
                for si in range(4):
                    ix = lax.shift_right_logical(x, shifts[si])
                    iy = lax.shift_right_logical(y, shifts[si])
                    iid = iy * ws[si] + ix
                    iid = jnp.where(m, iid, padrow[si])
                    idxs[si][pl.ds(j * L, L)] = iid
            cps = [pltpu.async_copy(tables[si].at[idxs[si]], rows[si], sem)
                   for si in range(4)]
            for cp in cps:
                cp.wait()

            def acc(r, _):
                for u in range(ROW_UNROLL):
                    rr = r * ROW_UNROLL + u
                    for k in range(cvec):
                        slk = pl.ds(k * L, L)
                        v = ((r0[rr, slk] + r1[rr, slk])
                             + (r2[rr, slk] + r3[rr, slk]))
                        r0[rr, slk] = v
                return _

            lax.fori_loop(0, BLK // ROW_UNROLL, acc, 0)
            pltpu.sync_copy(r0, outh.at[pl.ds(qbase + b * BLK, BLK)])
            return carry

        lax.fori_loop(0, nblk, block, 0)

    run = pl.kernel(
        body,
        out_type=jax.ShapeDtypeStruct((nq, c), jnp.float32),
        mesh=mesh,
        compiler_params=pltpu.CompilerParams(use_tc_tiling_on_sc=False),
        scratch_types=[
            pltpu.VMEM((qpw,), jnp.int32),
            pltpu.VMEM((qpw,), jnp.int32),
            pltpu.VMEM((qpw,), jnp.int32),
            pltpu.VMEM((BLK,), jnp.int32),
            pltpu.VMEM((BLK,), jnp.int32),
            pltpu.VMEM((BLK,), jnp.int32),
            pltpu.VMEM((BLK,), jnp.int32),
            pltpu.VMEM((BLK, c), jnp.float32),
            pltpu.VMEM((BLK, c), jnp.float32),
            pltpu.VMEM((BLK, c), jnp.float32),
            pltpu.VMEM((BLK, c), jnp.float32),
            pltpu.SemaphoreType.DMA,
        ],
    )
    return run(t1, t2, t4, t8, px, py, fov)


def kernel(feat_s1, feat_s2, feat_s4, feat_s8, projected_pix, fov_mask):
    feats = (feat_s1, feat_s2, feat_s4, feat_s8)
    bs, num_cam, c, h, w = feat_s1.shape
    nq = projected_pix.shape[1]

    tables = []
    for f in feats:
        hw = f.shape[3] * f.shape[4]
        t = f.reshape(c, hw).T
        tables.append(jnp.concatenate([t, jnp.zeros((1, c), t.dtype)], axis=0))

    px = projected_pix[0, :, 0]
    py = projected_pix[0, :, 1]
    fov = fov_mask[0].astype(jnp.int32)

    y = _flosp_sc(*tables, px, py, fov, nq=nq, c=c, h=h, w=w)
    return y.T.reshape(bs, c, nq)
